# Initial kernel scaffold; baseline (speedup 1.0000x reference)
import jax
import jax.numpy as jnp
from jax.experimental import pallas as pl


def kernel(subword_sequences, token_embedding):
    raise NotImplementedError("write your pallas kernel here")


# speedup vs baseline: 1.3788x; 1.3788x over previous
import functools

import jax
import jax.numpy as jnp
from jax import lax
from jax.experimental import pallas as pl
from jax.experimental.pallas import tpu as pltpu
from jax.experimental.pallas import tpu_sc as plsc

_B, _L, _EMBED = 16, 2048, 128
_TOTAL = _B * _L
_NC, _NS = 2, 16
_NW = _NC * _NS
_PER_W = _TOTAL // _NW
_CHUNK = 128
_NCHUNK = _PER_W // _CHUNK

_mesh = plsc.VectorSubcoreMesh(core_axis_name="c", subcore_axis_name="s")


@functools.partial(
    pl.kernel,
    mesh=_mesh,
    out_type=jax.ShapeDtypeStruct((_TOTAL, _EMBED), jnp.float32),
    scratch_types=[
        pltpu.VMEM((_NCHUNK, _CHUNK), jnp.int32),
        pltpu.VMEM((_CHUNK, _EMBED), jnp.float32),
        pltpu.VMEM((_CHUNK, _EMBED), jnp.float32),
        pltpu.SemaphoreType.DMA,
        pltpu.SemaphoreType.DMA,
    ],
)
def _gather_all(idx_hbm, table_hbm, out_hbm, idx_v, rows0, rows1, sem0, sem1):
    wid = lax.axis_index("s") * _NC + lax.axis_index("c")
    base = wid * _PER_W
    pltpu.sync_copy(idx_hbm.at[wid], idx_v)

    bufs = (rows0, rows1)
    sems = (sem0, sem1)
    copies = [None, None]
    copies[0] = pltpu.async_copy(table_hbm.at[idx_v.at[0]], bufs[0], sems[0])
    for j in range(1, _NCHUNK + 1):
        if j < _NCHUNK:
            copies[j % 2] = pltpu.async_copy(
                table_hbm.at[idx_v.at[j]], bufs[j % 2], sems[j % 2]
            )
        copies[(j - 1) % 2].wait()
        pltpu.sync_copy(
            bufs[(j - 1) % 2],
            out_hbm.at[pl.ds(base + (j - 1) * _CHUNK, _CHUNK)],
        )


def kernel(subword_sequences, token_embedding):
    idx = subword_sequences.astype(jnp.int32).reshape(_NW, _NCHUNK, _CHUNK)
    out = _gather_all(idx, token_embedding)
    return out.reshape(_B, _L, _EMBED)


# speedup vs baseline: 1.4324x; 1.0389x over previous
import functools

import jax
import jax.numpy as jnp
from jax import lax
from jax.experimental import pallas as pl
from jax.experimental.pallas import tpu as pltpu
from jax.experimental.pallas import tpu_sc as plsc

_B, _L, _EMBED = 16, 2048, 128
_TOTAL = _B * _L
_NC, _NS = 2, 16
_NW = _NC * _NS
_PER_W = _TOTAL // _NW
_CHUNK = 128
_NCHUNK = _PER_W // _CHUNK

_NBUF = 4

_mesh = plsc.VectorSubcoreMesh(core_axis_name="c", subcore_axis_name="s")


@functools.partial(
    pl.kernel,
    mesh=_mesh,
    out_type=jax.ShapeDtypeStruct((_TOTAL, _EMBED), jnp.float32),
    scratch_types=[
        pltpu.VMEM((_NCHUNK, _CHUNK), jnp.int32),
    ]
    + [pltpu.VMEM((_CHUNK, _EMBED), jnp.float32) for _ in range(_NBUF)]
    + [pltpu.SemaphoreType.DMA for _ in range(2 * _NBUF)],
)
def _gather_all(idx_hbm, table_hbm, out_hbm, idx_v, *scratch):
    bufs = scratch[:_NBUF]
    gsems = scratch[_NBUF : 2 * _NBUF]
    wsems = scratch[2 * _NBUF :]
    wid = lax.axis_index("s") * _NC + lax.axis_index("c")
    base = wid * _PER_W
    pltpu.sync_copy(idx_hbm.at[wid], idx_v)

    gcopies = [None] * _NBUF
    wcopies = [None] * _NBUF
    for j in range(_NBUF):
        gcopies[j] = pltpu.async_copy(
            table_hbm.at[idx_v.at[j]], bufs[j], gsems[j]
        )
    for j in range(_NCHUNK):
        b = j % _NBUF
        gcopies[b].wait()
        wcopies[b] = pltpu.async_copy(
            bufs[b], out_hbm.at[pl.ds(base + j * _CHUNK, _CHUNK)], wsems[b]
        )
        nxt = j + _NBUF
        if nxt < _NCHUNK:
            wcopies[b].wait()
            gcopies[b] = pltpu.async_copy(
                table_hbm.at[idx_v.at[nxt]], bufs[b], gsems[b]
            )
    for j in range(_NCHUNK - _NBUF, _NCHUNK):
        wcopies[j % _NBUF].wait()


def kernel(subword_sequences, token_embedding):
    idx = subword_sequences.astype(jnp.int32).reshape(_NW, _NCHUNK, _CHUNK)
    out = _gather_all(idx, token_embedding)
    return out.reshape(_B, _L, _EMBED)


# speedup vs baseline: 1.4572x; 1.0173x over previous
import functools

import jax
import jax.numpy as jnp
from jax import lax
from jax.experimental import pallas as pl
from jax.experimental.pallas import tpu as pltpu
from jax.experimental.pallas import tpu_sc as plsc

_B, _L, _EMBED = 16, 2048, 128
_TOTAL = _B * _L
_NC, _NS = 2, 16
_NW = _NC * _NS
_PER_W = _TOTAL // _NW
_CHUNK = 128
_NCHUNK = _PER_W // _CHUNK

_NBUF = 7

_mesh = plsc.VectorSubcoreMesh(core_axis_name="c", subcore_axis_name="s")


@functools.partial(
    pl.kernel,
    mesh=_mesh,
    out_type=jax.ShapeDtypeStruct((_TOTAL, _EMBED), jnp.float32),
    scratch_types=[
        pltpu.VMEM((_NCHUNK, _CHUNK), jnp.int32),
    ]
    + [pltpu.VMEM((_CHUNK, _EMBED), jnp.float32) for _ in range(_NBUF)]
    + [pltpu.SemaphoreType.DMA for _ in range(2 * _NBUF)],
)
def _gather_all(idx_hbm, table_hbm, out_hbm, idx_v, *scratch):
    bufs = scratch[:_NBUF]
    gsems = scratch[_NBUF : 2 * _NBUF]
    wsems = scratch[2 * _NBUF :]
    wid = lax.axis_index("s") * _NC + lax.axis_index("c")
    base = wid * _PER_W
    pltpu.sync_copy(idx_hbm.at[wid], idx_v)

    gcopies = [None] * _NBUF
    wcopies = [None] * _NBUF
    for j in range(_NBUF):
        gcopies[j] = pltpu.async_copy(
            table_hbm.at[idx_v.at[j]], bufs[j], gsems[j]
        )
    for j in range(_NCHUNK):
        b = j % _NBUF
        gcopies[b].wait()
        wcopies[b] = pltpu.async_copy(
            bufs[b], out_hbm.at[pl.ds(base + j * _CHUNK, _CHUNK)], wsems[b]
        )
        nxt = j + _NBUF
        if nxt < _NCHUNK:
            wcopies[b].wait()
            gcopies[b] = pltpu.async_copy(
                table_hbm.at[idx_v.at[nxt]], bufs[b], gsems[b]
            )
    for j in range(_NCHUNK - _NBUF, _NCHUNK):
        wcopies[j % _NBUF].wait()


def kernel(subword_sequences, token_embedding):
    idx = subword_sequences.astype(jnp.int32).reshape(_NW, _NCHUNK, _CHUNK)
    out = _gather_all(idx, token_embedding)
    return out.reshape(_B, _L, _EMBED)
